# baseline (speedup 1.0000x reference)
import jax
import jax.numpy as jnp
from jax.experimental import pallas as pl

NUM_EMB = 1024
EMB_DIM = 64
SPARSITY = 5
COMMIT = 0.25
EPS = 1e-10

B_TOK = 2048


def _omp_block_kernel(x_ref, d_ref, coef_ref, zst_ref, loss_ref):
    dict_raw = d_ref[...]
    norm = jnp.sqrt(jnp.sum(dict_raw * dict_raw, axis=0, keepdims=True))
    dn = dict_raw / norm

    half = B_TOK // 2
    x = x_ref[...]
    xs = [x[:, :half], x[:, half:]]
    ress = list(xs)
    pickss = [[], []]
    row_iota = jax.lax.broadcasted_iota(jnp.int32, (NUM_EMB, half), 0)

    d_hi = dn.astype(jnp.bfloat16)
    r1 = dn - d_hi.astype(jnp.float32)
    d_mid = r1.astype(jnp.bfloat16)
    d_lo = (r1 - d_mid.astype(jnp.float32)).astype(jnp.bfloat16)
    d_split = jnp.concatenate([d_hi, d_mid, d_lo], axis=0)

    for _ in range(SPARSITY):
        corrs = [jax.lax.dot_general(
            r, dn, (((0,), (0,)), ((), ())),
            preferred_element_type=jnp.float32) for r in ress]
        idxs = [jnp.argmax(jnp.abs(c), axis=1)[None, :] for c in corrs]
        onehots = [(row_iota == ix).astype(jnp.bfloat16) for ix in idxs]
        gs = [jax.lax.dot_general(
            d_split, oh, (((1,), (0,)), ((), ())),
            preferred_element_type=jnp.float32) for oh in onehots]
        d_sels = [(g[0:EMB_DIM] + g[EMB_DIM:2 * EMB_DIM]) + g[2 * EMB_DIM:]
                  for g in gs]
        for h in range(2):
            num = jnp.sum(ress[h] * d_sels[h], axis=0, keepdims=True)
            den = jnp.sum(d_sels[h] * d_sels[h], axis=0, keepdims=True)
            alpha = num / (den + EPS)
            pickss[h].append((idxs[h], alpha))
            ress[h] = ress[h] - d_sels[h] * alpha

    for h, lo in ((0, 0), (1, half)):
        picks = pickss[h]
        coef = jnp.where(row_iota == picks[0][0], picks[0][1], 0.0)
        for idx_t, alpha_t in picks[1:]:
            coef = coef + jnp.where(row_iota == idx_t, alpha_t, 0.0)
        coef_ref[:, lo:lo + half] = coef
        zst_ref[:, lo:lo + half] = xs[h] - ress[h]

    s = (jnp.sum(ress[0] * ress[0]) + jnp.sum(ress[1] * ress[1])).reshape(1, 1)
    i = pl.program_id(0)
    nblocks = pl.num_programs(0)

    @pl.when(i == 0)
    def _init():
        loss_ref[...] = s

    @pl.when(i != 0)
    def _acc():
        loss_ref[...] = loss_ref[...] + s

    @pl.when(i == nblocks - 1)
    def _finish():
        total = nblocks * EMB_DIM * B_TOK
        loss_ref[...] = loss_ref[...] * ((1.0 + COMMIT) / total)


def kernel(z_e, dictionary):
    z = jnp.transpose(z_e, (0, 2, 3, 1))
    input_shape = z.shape
    zf = z.reshape(EMB_DIM, -1)
    n_tok = zf.shape[1]
    grid = n_tok // B_TOK

    coef, zst, loss = pl.pallas_call(
        _omp_block_kernel,
        grid=(grid,),
        in_specs=[
            pl.BlockSpec((EMB_DIM, B_TOK), lambda i: (0, i)),
            pl.BlockSpec((EMB_DIM, NUM_EMB), lambda i: (0, 0)),
        ],
        out_specs=[
            pl.BlockSpec((NUM_EMB, B_TOK), lambda i: (0, i)),
            pl.BlockSpec((EMB_DIM, B_TOK), lambda i: (0, i)),
            pl.BlockSpec((1, 1), lambda i: (0, 0)),
        ],
        out_shape=[
            jax.ShapeDtypeStruct((NUM_EMB, n_tok), jnp.float32),
            jax.ShapeDtypeStruct((EMB_DIM, n_tok), jnp.float32),
            jax.ShapeDtypeStruct((1, 1), jnp.float32),
        ],
    )(zf, dictionary)

    z_st = jnp.transpose(zst.reshape(input_shape), (0, 3, 1, 2))
    return (z_st, loss[0, 0], coef)


# speedup vs baseline: 1.1537x; 1.1537x over previous
import jax
import jax.numpy as jnp
from jax.experimental import pallas as pl

NUM_EMB = 1024
EMB_DIM = 64
SPARSITY = 5
COMMIT = 0.25
EPS = 1e-10

B_TOK = 2048


def _omp_block_kernel(x_ref, d_ref, coef_ref, zst_ref, loss_ref):
    dict_raw = d_ref[...]
    norm = jnp.sqrt(jnp.sum(dict_raw * dict_raw, axis=0, keepdims=True))
    dn = dict_raw / norm

    half = B_TOK // 2
    x = x_ref[...]
    xs = [x[:, :half], x[:, half:]]
    ress = list(xs)
    pickss = [[], []]
    row_iota = jax.lax.broadcasted_iota(jnp.int32, (NUM_EMB, half), 0)

    d_hi = dn.astype(jnp.bfloat16)
    r1 = dn - d_hi.astype(jnp.float32)
    d_mid = r1.astype(jnp.bfloat16)
    d_lo = (r1 - d_mid.astype(jnp.float32)).astype(jnp.bfloat16)
    d_split = jnp.concatenate([d_hi, d_mid, d_lo], axis=0)

    for _ in range(SPARSITY):
        corrs = [jax.lax.dot_general(
            dn, r, (((0,), (0,)), ((), ())),
            preferred_element_type=jnp.float32) for r in ress]
        idxs = [jnp.argmax(jnp.abs(c), axis=0)[None, :] for c in corrs]
        onehots = [(row_iota == ix).astype(jnp.bfloat16) for ix in idxs]
        gs = [jax.lax.dot_general(
            d_split, oh, (((1,), (0,)), ((), ())),
            preferred_element_type=jnp.float32) for oh in onehots]
        d_sels = [(g[0:EMB_DIM] + g[EMB_DIM:2 * EMB_DIM]) + g[2 * EMB_DIM:]
                  for g in gs]
        for h in range(2):
            num = jnp.sum(ress[h] * d_sels[h], axis=0, keepdims=True)
            den = jnp.sum(d_sels[h] * d_sels[h], axis=0, keepdims=True)
            alpha = num / (den + EPS)
            pickss[h].append((idxs[h], alpha))
            ress[h] = ress[h] - d_sels[h] * alpha

    for h, lo in ((0, 0), (1, half)):
        picks = pickss[h]
        coef = jnp.where(row_iota == picks[0][0], picks[0][1], 0.0)
        for idx_t, alpha_t in picks[1:]:
            coef = coef + jnp.where(row_iota == idx_t, alpha_t, 0.0)
        coef_ref[:, lo:lo + half] = coef
        zst_ref[:, lo:lo + half] = xs[h] - ress[h]

    s = (jnp.sum(ress[0] * ress[0]) + jnp.sum(ress[1] * ress[1])).reshape(1, 1)
    i = pl.program_id(0)
    nblocks = pl.num_programs(0)

    @pl.when(i == 0)
    def _init():
        loss_ref[...] = s

    @pl.when(i != 0)
    def _acc():
        loss_ref[...] = loss_ref[...] + s

    @pl.when(i == nblocks - 1)
    def _finish():
        total = nblocks * EMB_DIM * B_TOK
        loss_ref[...] = loss_ref[...] * ((1.0 + COMMIT) / total)


def kernel(z_e, dictionary):
    z = jnp.transpose(z_e, (0, 2, 3, 1))
    input_shape = z.shape
    zf = z.reshape(EMB_DIM, -1)
    n_tok = zf.shape[1]
    grid = n_tok // B_TOK

    coef, zst, loss = pl.pallas_call(
        _omp_block_kernel,
        grid=(grid,),
        in_specs=[
            pl.BlockSpec((EMB_DIM, B_TOK), lambda i: (0, i)),
            pl.BlockSpec((EMB_DIM, NUM_EMB), lambda i: (0, 0)),
        ],
        out_specs=[
            pl.BlockSpec((NUM_EMB, B_TOK), lambda i: (0, i)),
            pl.BlockSpec((EMB_DIM, B_TOK), lambda i: (0, i)),
            pl.BlockSpec((1, 1), lambda i: (0, 0)),
        ],
        out_shape=[
            jax.ShapeDtypeStruct((NUM_EMB, n_tok), jnp.float32),
            jax.ShapeDtypeStruct((EMB_DIM, n_tok), jnp.float32),
            jax.ShapeDtypeStruct((1, 1), jnp.float32),
        ],
    )(zf, dictionary)

    z_st = jnp.transpose(zst.reshape(input_shape), (0, 3, 1, 2))
    return (z_st, loss[0, 0], coef)


# speedup vs baseline: 1.1554x; 1.0015x over previous
import jax
import jax.numpy as jnp
from jax.experimental import pallas as pl

NUM_EMB = 1024
EMB_DIM = 64
SPARSITY = 5
COMMIT = 0.25
EPS = 1e-10

B_TOK = 2048


def _omp_block_kernel(x_ref, d_ref, coef_ref, zst_ref, loss_ref):
    dict_raw = d_ref[...]
    norm = jnp.sqrt(jnp.sum(dict_raw * dict_raw, axis=0, keepdims=True))
    dn = dict_raw / norm

    nsplit = 4
    half = B_TOK // nsplit
    x = x_ref[...]
    xs = [x[:, k * half:(k + 1) * half] for k in range(nsplit)]
    ress = list(xs)
    pickss = [[] for _ in range(nsplit)]
    row_iota = jax.lax.broadcasted_iota(jnp.int32, (NUM_EMB, half), 0)

    d_hi = dn.astype(jnp.bfloat16)
    r1 = dn - d_hi.astype(jnp.float32)
    d_mid = r1.astype(jnp.bfloat16)
    d_lo = (r1 - d_mid.astype(jnp.float32)).astype(jnp.bfloat16)
    d_split = jnp.concatenate([d_hi, d_mid, d_lo], axis=0)

    for _ in range(SPARSITY):
        corrs = [jax.lax.dot_general(
            dn, r, (((0,), (0,)), ((), ())),
            preferred_element_type=jnp.float32) for r in ress]
        idxs = [jnp.argmax(jnp.abs(c), axis=0)[None, :] for c in corrs]
        onehots = [(row_iota == ix).astype(jnp.bfloat16) for ix in idxs]
        gs = [jax.lax.dot_general(
            d_split, oh, (((1,), (0,)), ((), ())),
            preferred_element_type=jnp.float32) for oh in onehots]
        d_sels = [(g[0:EMB_DIM] + g[EMB_DIM:2 * EMB_DIM]) + g[2 * EMB_DIM:]
                  for g in gs]
        for h in range(nsplit):
            num = jnp.sum(ress[h] * d_sels[h], axis=0, keepdims=True)
            den = jnp.sum(d_sels[h] * d_sels[h], axis=0, keepdims=True)
            alpha = num / (den + EPS)
            pickss[h].append((idxs[h], alpha))
            ress[h] = ress[h] - d_sels[h] * alpha

    for h in range(nsplit):
        lo = h * half
        picks = pickss[h]
        coef = jnp.where(row_iota == picks[0][0], picks[0][1], 0.0)
        for idx_t, alpha_t in picks[1:]:
            coef = coef + jnp.where(row_iota == idx_t, alpha_t, 0.0)
        coef_ref[:, lo:lo + half] = coef
        zst_ref[:, lo:lo + half] = xs[h] - ress[h]

    s = sum(jnp.sum(r * r) for r in ress).reshape(1, 1)
    i = pl.program_id(0)
    nblocks = pl.num_programs(0)

    @pl.when(i == 0)
    def _init():
        loss_ref[...] = s

    @pl.when(i != 0)
    def _acc():
        loss_ref[...] = loss_ref[...] + s

    @pl.when(i == nblocks - 1)
    def _finish():
        total = nblocks * EMB_DIM * B_TOK
        loss_ref[...] = loss_ref[...] * ((1.0 + COMMIT) / total)


def kernel(z_e, dictionary):
    z = jnp.transpose(z_e, (0, 2, 3, 1))
    input_shape = z.shape
    zf = z.reshape(EMB_DIM, -1)
    n_tok = zf.shape[1]
    grid = n_tok // B_TOK

    coef, zst, loss = pl.pallas_call(
        _omp_block_kernel,
        grid=(grid,),
        in_specs=[
            pl.BlockSpec((EMB_DIM, B_TOK), lambda i: (0, i)),
            pl.BlockSpec((EMB_DIM, NUM_EMB), lambda i: (0, 0)),
        ],
        out_specs=[
            pl.BlockSpec((NUM_EMB, B_TOK), lambda i: (0, i)),
            pl.BlockSpec((EMB_DIM, B_TOK), lambda i: (0, i)),
            pl.BlockSpec((1, 1), lambda i: (0, 0)),
        ],
        out_shape=[
            jax.ShapeDtypeStruct((NUM_EMB, n_tok), jnp.float32),
            jax.ShapeDtypeStruct((EMB_DIM, n_tok), jnp.float32),
            jax.ShapeDtypeStruct((1, 1), jnp.float32),
        ],
    )(zf, dictionary)

    z_st = jnp.transpose(zst.reshape(input_shape), (0, 3, 1, 2))
    return (z_st, loss[0, 0], coef)
